# baseline (speedup 1.0000x reference)
import functools

import jax
import jax.numpy as jnp
from jax import lax
from jax.experimental import pallas as pl
from jax.experimental.pallas import tpu as pltpu
from jax.experimental.pallas import tpu_sc as plsc

_TOPK_FRAC = 0.25

_ROWS = 8192
_CHUNK = 1024
_NBUF = 4
_NCH = 16
_NB1 = 2048
_NB2 = 1024
_UNROLL = 8


def _sc_body(x_hbm, o_hbm, bufs, hist1, hist2, isems, osems, *,
             k, d_off, d_span, groups_per_worker):
    nchunks = _ROWS // _CHUNK
    groups_per_batch = d_span // _NCH
    iota = lax.iota(jnp.int32, 16)
    ones = jnp.ones((16,), jnp.int32)
    zeros16 = jnp.zeros((16,), jnp.int32)
    i31 = jnp.int32(0x7FFFFFFF)
    kv0 = jnp.full((16,), k, jnp.int32)

    ncores = 2
    wid = lax.axis_index("s") * ncores + lax.axis_index("c")

    def zero_hist(ref, nbins):
        @plsc.parallel_loop(0, nbins, unroll=_UNROLL)
        def zb(i):
            ref[pl.ds(i * 16, 16)] = zeros16

    zero_hist(hist1, _NB1)
    zero_hist(hist2, _NB2)

    def scan_hist(ref, nbins, kv):
        def sb(i, carry):
            acc, bsel, krem = carry
            b = nbins - 1 - i
            h = ref[pl.ds(b * 16, 16)]
            ref[pl.ds(b * 16, 16)] = zeros16
            nacc = acc + h
            crossed = jnp.logical_and(acc < kv, nacc >= kv)
            bsel = jnp.where(crossed, b, bsel)
            krem = jnp.where(crossed, kv - acc, krem)
            return nacc, bsel, krem
        _, bsel, krem = lax.fori_loop(
            0, nbins, sb, (zeros16, zeros16, zeros16), unroll=4)
        return bsel, krem

    def group_body(gi, carry):
        g = wid * groups_per_worker + gi
        n = g // groups_per_batch
        dd = (g % groups_per_batch) * _NCH

        def issue_in(c, slot):
            return pltpu.async_copy(
                x_hbm.at[n, pl.ds(c * _CHUNK, _CHUNK), pl.ds(d_off + dd, _NCH)],
                bufs[slot], isems[slot])

        def issue_out(c, slot):
            return pltpu.async_copy(
                bufs[slot],
                o_hbm.at[n, pl.ds(c * _CHUNK, _CHUNK), pl.ds(dd, _NCH)],
                osems[slot])

        def stream_pass(resident, row_body):
            rest = [c for c in range(nchunks) if c not in resident]
            order = list(resident) + rest
            hs = [None] * nchunks
            for p in range(len(resident), min(_NBUF, nchunks)):
                hs[p] = issue_in(order[p], p % _NBUF)
            for p in range(nchunks):
                if hs[p] is not None:
                    hs[p].wait()
                b = bufs[p % _NBUF]

                @plsc.parallel_loop(0, _CHUNK, unroll=_UNROLL)
                def row(r, b=b):
                    row_body(b, r)
                q = p + _NBUF
                if q < nchunks:
                    hs[q] = issue_in(order[q], q % _NBUF)
            return order[-_NBUF:]

        def p1(b, r):
            a = lax.bitcast_convert_type(b[r], jnp.int32) & i31
            idx = ((a >> 16) & jnp.int32(0x7FF0)) | iota
            plsc.addupdate_scatter(hist1, [idx], ones)
        resident = stream_pass([], p1)
        b1, k1 = scan_hist(hist1, _NB1, kv0)

        def p2(b, r):
            a = lax.bitcast_convert_type(b[r], jnp.int32) & i31
            m = (a >> 20) == b1
            idx = ((a >> 6) & jnp.int32(0x3FF0)) | iota
            plsc.addupdate_scatter(hist2, [idx], ones, mask=m)
        resident = stream_pass(resident, p2)
        b2, k2 = scan_hist(hist2, _NB2, k1)

        pfx = (b1 << 10) | b2

        def p3(b, r):
            a = lax.bitcast_convert_type(b[r], jnp.int32) & i31
            m = (a >> 10) == pfx
            idx = ((a << 4) & jnp.int32(0x3FF0)) | iota
            plsc.addupdate_scatter(hist2, [idx], ones, mask=m)
        resident = stream_pass(resident, p3)
        b3, _ = scan_hist(hist2, _NB2, k2)
        t = (pfx << 10) | b3

        rest = [c for c in range(nchunks) if c not in resident]
        order = list(resident) + rest
        hs = [None] * nchunks
        out_h = [None] * nchunks
        waited = set()
        for p in range(nchunks):
            if hs[p] is not None:
                hs[p].wait()
            b = bufs[p % _NBUF]

            @plsc.parallel_loop(0, _CHUNK, unroll=_UNROLL)
            def row(r, b=b):
                v = b[r]
                a = lax.bitcast_convert_type(v, jnp.int32) & i31
                b[r] = jnp.where(a >= t, v, jnp.float32(0.0))
            out_h[p] = issue_out(order[p], p % _NBUF)
            q = p + 2
            if q >= len(resident) and q < nchunks and hs[q] is None:
                out_h[q - _NBUF].wait()
                waited.add(q - _NBUF)
                hs[q] = issue_in(order[q], q % _NBUF)
        for p in range(nchunks):
            if p not in waited:
                out_h[p].wait()
        return carry

    lax.fori_loop(0, groups_per_worker, group_body, 0)


def _tc_select_body(x_ref, o_ref, *, k):
    xv = x_ref[0]
    bits = jax.lax.bitcast_convert_type(xv, jnp.int32) & jnp.int32(0x7FFFFFFF)
    hi = jnp.max(bits, axis=0, keepdims=True)
    lo = jnp.zeros_like(hi)

    def step(_, carry):
        lo, hi = carry
        mid = lo + (hi - lo + 1) // 2
        cnt = jnp.sum((bits >= mid).astype(jnp.int32), axis=0, keepdims=True)
        ge = cnt >= k
        return jnp.where(ge, mid, lo), jnp.where(ge, hi, mid - 1)

    lo, hi = jax.lax.fori_loop(0, 31, step, (lo, hi))
    o_ref[0] = jnp.where(bits >= lo, xv, jnp.float32(0.0))


def _sc_part(x, k, d_off, d_span):
    n, hw1, d = x.shape
    nworkers = 32
    groups_per_worker = (n * d_span) // _NCH // nworkers
    mesh = plsc.VectorSubcoreMesh(core_axis_name="c", subcore_axis_name="s")

    def body(x_hbm, o_hbm, b0, b1, b2, b3, hist1, hist2,
             i0, i1, i2, i3, o0, o1, o2, o3):
        _sc_body(x_hbm, o_hbm, (b0, b1, b2, b3), hist1, hist2,
                 (i0, i1, i2, i3), (o0, o1, o2, o3),
                 k=k, d_off=d_off, d_span=d_span,
                 groups_per_worker=groups_per_worker)

    f = pl.kernel(
        body,
        mesh=mesh,
        out_type=jax.ShapeDtypeStruct((n, hw1, d_span), x.dtype),
        scratch_types=(
            [pltpu.VMEM((_CHUNK, _NCH), jnp.float32) for _ in range(_NBUF)]
            + [pltpu.VMEM((_NB1 * 16,), jnp.int32),
               pltpu.VMEM((_NB2 * 16,), jnp.int32)]
            + [pltpu.SemaphoreType.DMA for _ in range(2 * _NBUF)]
        ),
        compiler_params=pltpu.CompilerParams(
            use_tc_tiling_on_sc=False, needs_layout_passes=False),
    )
    return f(x)


def _tc_part(x, k, d_span):
    n, hw1, d = x.shape
    cblk = min(d_span, 256)
    return pl.pallas_call(
        functools.partial(_tc_select_body, k=k),
        grid=(n, d_span // cblk),
        in_specs=[pl.BlockSpec((1, hw1, cblk), lambda i, j: (i, 0, j))],
        out_specs=pl.BlockSpec((1, hw1, cblk), lambda i, j: (i, 0, j)),
        out_shape=jax.ShapeDtypeStruct((n, hw1, d_span), x.dtype),
        compiler_params=pltpu.CompilerParams(
            dimension_semantics=("parallel", "parallel"),
        ),
    )(x)


def _concat_body(a_ref, b_ref, o_ref, *, ja):
    j = pl.program_id(1)

    @pl.when(j < ja)
    def _():
        o_ref[...] = a_ref[...]

    @pl.when(j >= ja)
    def _():
        o_ref[...] = b_ref[...]


def _tc_concat(a, b):
    n, hw1, da = a.shape
    db = b.shape[2]
    cblk = 256
    ja, jb = da // cblk, db // cblk
    return pl.pallas_call(
        functools.partial(_concat_body, ja=ja),
        grid=(n, ja + jb),
        in_specs=[
            pl.BlockSpec((1, hw1, cblk),
                         lambda i, j: (i, 0, jnp.minimum(j, ja - 1))),
            pl.BlockSpec((1, hw1, cblk),
                         lambda i, j: (i, 0, jnp.maximum(j - ja, 0))),
        ],
        out_specs=pl.BlockSpec((1, hw1, cblk), lambda i, j: (i, 0, j)),
        out_shape=jax.ShapeDtypeStruct((n, hw1, da + db), a.dtype),
        compiler_params=pltpu.CompilerParams(
            dimension_semantics=("parallel", "arbitrary"),
        ),
    )(a, b)


_TC_SPAN = 512


def kernel(x):
    n, hw1, d = x.shape
    k = max(1, int(hw1 * _TOPK_FRAC))
    tc_span = _TC_SPAN if 0 < _TC_SPAN < d else 0
    sc_span = d - tc_span
    if tc_span == 0:
        return _sc_part(x, k, 0, sc_span)
    tc_out = _tc_part(x, k, tc_span)
    sc_out = _sc_part(x, k, tc_span, sc_span)
    return _tc_concat(tc_out, sc_out)


# speedup vs baseline: 1.0817x; 1.0817x over previous
import functools

import jax
import jax.numpy as jnp
from jax import lax
from jax.experimental import pallas as pl
from jax.experimental.pallas import tpu as pltpu
from jax.experimental.pallas import tpu_sc as plsc

_TOPK_FRAC = 0.25

_ROWS = 8192
_CHUNK = 1024
_NBUF = 4
_NCH = 16
_NB1 = 2048
_NB2 = 1024
_UNROLL = 8


def _sc_body(x_hbm, o_hbm, bufs, hist1, hist2, isems, osems, *,
             k, d_off, d_span, groups_per_worker):
    nchunks = _ROWS // _CHUNK
    groups_per_batch = d_span // _NCH
    iota = lax.iota(jnp.int32, 16)
    ones = jnp.ones((16,), jnp.int32)
    zeros16 = jnp.zeros((16,), jnp.int32)
    i31 = jnp.int32(0x7FFFFFFF)
    kv0 = jnp.full((16,), k, jnp.int32)

    ncores = 2
    wid = lax.axis_index("s") * ncores + lax.axis_index("c")

    def zero_hist(ref, nbins):
        @plsc.parallel_loop(0, nbins, unroll=_UNROLL)
        def zb(i):
            ref[pl.ds(i * 16, 16)] = zeros16

    zero_hist(hist1, _NB1)
    zero_hist(hist2, _NB2)

    def scan_hist(ref, nbins, kv):
        def sb(i, carry):
            acc, bsel, krem = carry
            b = nbins - 1 - i
            h = ref[pl.ds(b * 16, 16)]
            ref[pl.ds(b * 16, 16)] = zeros16
            nacc = acc + h
            crossed = jnp.logical_and(acc < kv, nacc >= kv)
            bsel = jnp.where(crossed, b, bsel)
            krem = jnp.where(crossed, kv - acc, krem)
            return nacc, bsel, krem
        _, bsel, krem = lax.fori_loop(
            0, nbins, sb, (zeros16, zeros16, zeros16), unroll=4)
        return bsel, krem

    def group_body(gi, carry):
        g = wid * groups_per_worker + gi
        n = g // groups_per_batch
        dd = (g % groups_per_batch) * _NCH

        def issue_in(c, slot):
            return pltpu.async_copy(
                x_hbm.at[n, pl.ds(c * _CHUNK, _CHUNK), pl.ds(d_off + dd, _NCH)],
                bufs[slot], isems[slot])

        def issue_out(c, slot):
            return pltpu.async_copy(
                bufs[slot],
                o_hbm.at[n, pl.ds(c * _CHUNK, _CHUNK), pl.ds(dd, _NCH)],
                osems[slot])

        def stream_pass(resident, row_body):
            rest = [c for c in range(nchunks) if c not in resident]
            order = list(resident) + rest
            hs = [None] * nchunks
            for p in range(len(resident), min(_NBUF, nchunks)):
                hs[p] = issue_in(order[p], p % _NBUF)
            for p in range(nchunks):
                if hs[p] is not None:
                    hs[p].wait()
                b = bufs[p % _NBUF]

                @plsc.parallel_loop(0, _CHUNK, unroll=_UNROLL)
                def row(r, b=b):
                    row_body(b, r)
                q = p + _NBUF
                if q < nchunks:
                    hs[q] = issue_in(order[q], q % _NBUF)
            return order[-_NBUF:]

        def p1(b, r):
            a = lax.bitcast_convert_type(b[r], jnp.int32) & i31
            idx = ((a >> 16) & jnp.int32(0x7FF0)) | iota
            plsc.addupdate_scatter(hist1, [idx], ones)
        resident = stream_pass([], p1)
        b1, k1 = scan_hist(hist1, _NB1, kv0)

        def p2(b, r):
            a = lax.bitcast_convert_type(b[r], jnp.int32) & i31
            m = (a >> 20) == b1
            idx = ((a >> 6) & jnp.int32(0x3FF0)) | iota
            plsc.addupdate_scatter(hist2, [idx], ones, mask=m)
        resident = stream_pass(resident, p2)
        b2, k2 = scan_hist(hist2, _NB2, k1)

        pfx = (b1 << 10) | b2

        def p3(b, r):
            a = lax.bitcast_convert_type(b[r], jnp.int32) & i31
            m = (a >> 10) == pfx
            idx = ((a << 4) & jnp.int32(0x3FF0)) | iota
            plsc.addupdate_scatter(hist2, [idx], ones, mask=m)
        resident = stream_pass(resident, p3)
        b3, _ = scan_hist(hist2, _NB2, k2)
        t = (pfx << 10) | b3

        rest = [c for c in range(nchunks) if c not in resident]
        order = list(resident) + rest
        hs = [None] * nchunks
        out_h = [None] * nchunks
        waited = set()
        for p in range(nchunks):
            if hs[p] is not None:
                hs[p].wait()
            b = bufs[p % _NBUF]

            @plsc.parallel_loop(0, _CHUNK, unroll=_UNROLL)
            def row(r, b=b):
                v = b[r]
                a = lax.bitcast_convert_type(v, jnp.int32) & i31
                b[r] = jnp.where(a >= t, v, jnp.float32(0.0))
            out_h[p] = issue_out(order[p], p % _NBUF)
            q = p + 2
            if q >= len(resident) and q < nchunks and hs[q] is None:
                out_h[q - _NBUF].wait()
                waited.add(q - _NBUF)
                hs[q] = issue_in(order[q], q % _NBUF)
        for p in range(nchunks):
            if p not in waited:
                out_h[p].wait()
        return carry

    lax.fori_loop(0, groups_per_worker, group_body, 0)


def _tc_select_body(x_ref, o_ref, *, k):
    xv = x_ref[0]
    bits = jax.lax.bitcast_convert_type(xv, jnp.int32) & jnp.int32(0x7FFFFFFF)
    hi = jnp.max(bits, axis=0, keepdims=True) >> 5
    lo = jnp.zeros_like(hi)

    def step(_, carry):
        lo, hi = carry
        mid = lo + (hi - lo + 1) // 2
        cnt = jnp.sum((bits >= (mid << 5)).astype(jnp.int32),
                      axis=0, keepdims=True)
        ge = cnt >= k
        return jnp.where(ge, mid, lo), jnp.where(ge, hi, mid - 1)

    lo, hi = jax.lax.fori_loop(0, 26, step, (lo, hi))
    o_ref[0] = jnp.where(bits >= (lo << 5), xv, jnp.float32(0.0))


def _sc_part(x, k, d_off, d_span):
    n, hw1, d = x.shape
    nworkers = 32
    groups_per_worker = (n * d_span) // _NCH // nworkers
    mesh = plsc.VectorSubcoreMesh(core_axis_name="c", subcore_axis_name="s")

    def body(x_hbm, o_hbm, b0, b1, b2, b3, hist1, hist2,
             i0, i1, i2, i3, o0, o1, o2, o3):
        _sc_body(x_hbm, o_hbm, (b0, b1, b2, b3), hist1, hist2,
                 (i0, i1, i2, i3), (o0, o1, o2, o3),
                 k=k, d_off=d_off, d_span=d_span,
                 groups_per_worker=groups_per_worker)

    f = pl.kernel(
        body,
        mesh=mesh,
        out_type=jax.ShapeDtypeStruct((n, hw1, d_span), x.dtype),
        scratch_types=(
            [pltpu.VMEM((_CHUNK, _NCH), jnp.float32) for _ in range(_NBUF)]
            + [pltpu.VMEM((_NB1 * 16,), jnp.int32),
               pltpu.VMEM((_NB2 * 16,), jnp.int32)]
            + [pltpu.SemaphoreType.DMA for _ in range(2 * _NBUF)]
        ),
        compiler_params=pltpu.CompilerParams(
            use_tc_tiling_on_sc=False, needs_layout_passes=False),
    )
    return f(x)


def _tc_part(x, k, d_span):
    n, hw1, d = x.shape
    cblk = min(d_span, 256)
    return pl.pallas_call(
        functools.partial(_tc_select_body, k=k),
        grid=(n, d_span // cblk),
        in_specs=[pl.BlockSpec((1, hw1, cblk), lambda i, j: (i, 0, j))],
        out_specs=pl.BlockSpec((1, hw1, cblk), lambda i, j: (i, 0, j)),
        out_shape=jax.ShapeDtypeStruct((n, hw1, d_span), x.dtype),
        compiler_params=pltpu.CompilerParams(
            dimension_semantics=("parallel", "parallel"),
        ),
    )(x)


def _concat_body(a_ref, b_ref, o_ref, *, ja):
    j = pl.program_id(1)

    @pl.when(j < ja)
    def _():
        o_ref[...] = a_ref[...]

    @pl.when(j >= ja)
    def _():
        o_ref[...] = b_ref[...]


def _tc_concat(a, b):
    n, hw1, da = a.shape
    db = b.shape[2]
    cblk = 256
    ja, jb = da // cblk, db // cblk
    return pl.pallas_call(
        functools.partial(_concat_body, ja=ja),
        grid=(n, ja + jb),
        in_specs=[
            pl.BlockSpec((1, hw1, cblk),
                         lambda i, j: (i, 0, jnp.minimum(j, ja - 1))),
            pl.BlockSpec((1, hw1, cblk),
                         lambda i, j: (i, 0, jnp.maximum(j - ja, 0))),
        ],
        out_specs=pl.BlockSpec((1, hw1, cblk), lambda i, j: (i, 0, j)),
        out_shape=jax.ShapeDtypeStruct((n, hw1, da + db), a.dtype),
        compiler_params=pltpu.CompilerParams(
            dimension_semantics=("parallel", "arbitrary"),
        ),
    )(a, b)


_TC_SPAN = 512


def kernel(x):
    n, hw1, d = x.shape
    k = max(1, int(hw1 * _TOPK_FRAC))
    tc_span = _TC_SPAN if 0 < _TC_SPAN < d else 0
    sc_span = d - tc_span
    if tc_span == 0:
        return _sc_part(x, k, 0, sc_span)
    tc_out = _tc_part(x, k, tc_span)
    sc_out = _sc_part(x, k, tc_span, sc_span)
    return _tc_concat(tc_out, sc_out)


# speedup vs baseline: 1.0894x; 1.0071x over previous
import functools

import jax
import jax.numpy as jnp
from jax import lax
from jax.experimental import pallas as pl
from jax.experimental.pallas import tpu as pltpu
from jax.experimental.pallas import tpu_sc as plsc

_TOPK_FRAC = 0.25

_ROWS = 8192
_CHUNK = 1024
_NBUF = 4
_NCH = 16
_NB1 = 2048
_NB2 = 1024
_UNROLL = 8


def _sc_body(x_hbm, o_hbm, bufs, hist1, hist2, isems, osems, *,
             k, d_off, d_span, groups_per_worker):
    nchunks = _ROWS // _CHUNK
    groups_per_batch = d_span // _NCH
    iota = lax.iota(jnp.int32, 16)
    ones = jnp.ones((16,), jnp.int32)
    zeros16 = jnp.zeros((16,), jnp.int32)
    i31 = jnp.int32(0x7FFFFFFF)
    kv0 = jnp.full((16,), k, jnp.int32)

    ncores = 2
    wid = lax.axis_index("s") * ncores + lax.axis_index("c")

    def zero_hist(ref, nbins):
        @plsc.parallel_loop(0, nbins, unroll=_UNROLL)
        def zb(i):
            ref[pl.ds(i * 16, 16)] = zeros16

    zero_hist(hist1, _NB1)
    zero_hist(hist2, _NB2)

    def scan_hist(ref, nbins, kv):
        def sb(i, carry):
            acc, bsel, krem = carry
            b = nbins - 1 - i
            h = ref[pl.ds(b * 16, 16)]
            ref[pl.ds(b * 16, 16)] = zeros16
            nacc = acc + h
            crossed = jnp.logical_and(acc < kv, nacc >= kv)
            bsel = jnp.where(crossed, b, bsel)
            krem = jnp.where(crossed, kv - acc, krem)
            return nacc, bsel, krem
        _, bsel, krem = lax.fori_loop(
            0, nbins, sb, (zeros16, zeros16, zeros16), unroll=4)
        return bsel, krem

    def group_body(gi, carry):
        g = wid * groups_per_worker + gi
        n = g // groups_per_batch
        dd = (g % groups_per_batch) * _NCH

        def issue_in(c, slot):
            return pltpu.async_copy(
                x_hbm.at[n, pl.ds(c * _CHUNK, _CHUNK), pl.ds(d_off + dd, _NCH)],
                bufs[slot], isems[slot])

        def issue_out(c, slot):
            return pltpu.async_copy(
                bufs[slot],
                o_hbm.at[n, pl.ds(c * _CHUNK, _CHUNK), pl.ds(dd, _NCH)],
                osems[slot])

        def stream_pass(resident, row_body):
            rest = [c for c in range(nchunks) if c not in resident]
            order = list(resident) + rest
            hs = [None] * nchunks
            for p in range(len(resident), min(_NBUF, nchunks)):
                hs[p] = issue_in(order[p], p % _NBUF)
            for p in range(nchunks):
                if hs[p] is not None:
                    hs[p].wait()
                b = bufs[p % _NBUF]

                @plsc.parallel_loop(0, _CHUNK, unroll=_UNROLL)
                def row(r, b=b):
                    row_body(b, r)
                q = p + _NBUF
                if q < nchunks:
                    hs[q] = issue_in(order[q], q % _NBUF)
            return order[-_NBUF:]

        def p1(b, r):
            a = lax.bitcast_convert_type(b[r], jnp.int32) & i31
            idx = ((a >> 16) & jnp.int32(0x7FF0)) | iota
            plsc.addupdate_scatter(hist1, [idx], ones)
        resident = stream_pass([], p1)
        b1, k1 = scan_hist(hist1, _NB1, kv0)

        def p2(b, r):
            a = lax.bitcast_convert_type(b[r], jnp.int32) & i31
            m = (a >> 20) == b1
            idx = ((a >> 6) & jnp.int32(0x3FF0)) | iota
            plsc.addupdate_scatter(hist2, [idx], ones, mask=m)
        resident = stream_pass(resident, p2)
        b2, k2 = scan_hist(hist2, _NB2, k1)

        pfx = (b1 << 10) | b2

        def p3(b, r):
            a = lax.bitcast_convert_type(b[r], jnp.int32) & i31
            m = (a >> 10) == pfx
            idx = ((a << 4) & jnp.int32(0x3FF0)) | iota
            plsc.addupdate_scatter(hist2, [idx], ones, mask=m)
        resident = stream_pass(resident, p3)
        b3, _ = scan_hist(hist2, _NB2, k2)
        t = (pfx << 10) | b3

        rest = [c for c in range(nchunks) if c not in resident]
        order = list(resident) + rest
        hs = [None] * nchunks
        out_h = [None] * nchunks
        waited = set()
        for p in range(nchunks):
            if hs[p] is not None:
                hs[p].wait()
            b = bufs[p % _NBUF]

            @plsc.parallel_loop(0, _CHUNK, unroll=_UNROLL)
            def row(r, b=b):
                v = b[r]
                a = lax.bitcast_convert_type(v, jnp.int32) & i31
                b[r] = jnp.where(a >= t, v, jnp.float32(0.0))
            out_h[p] = issue_out(order[p], p % _NBUF)
            q = p + 2
            if q >= len(resident) and q < nchunks and hs[q] is None:
                out_h[q - _NBUF].wait()
                waited.add(q - _NBUF)
                hs[q] = issue_in(order[q], q % _NBUF)
        for p in range(nchunks):
            if p not in waited:
                out_h[p].wait()
        return carry

    lax.fori_loop(0, groups_per_worker, group_body, 0)


def _tc_select_body(x_ref, o_ref, *, k):
    xv = x_ref[0]
    bits = jax.lax.bitcast_convert_type(xv, jnp.int32) & jnp.int32(0x7FFFFFFF)
    hi = jnp.max(bits, axis=0, keepdims=True) >> 5
    lo = jnp.zeros_like(hi)

    def step(_, carry):
        lo, hi = carry
        mid = lo + (hi - lo + 1) // 2
        cnt = jnp.sum((bits >= (mid << 5)).astype(jnp.int32),
                      axis=0, keepdims=True)
        ge = cnt >= k
        return jnp.where(ge, mid, lo), jnp.where(ge, hi, mid - 1)

    lo, hi = jax.lax.fori_loop(0, 26, step, (lo, hi))
    o_ref[0] = jnp.where(bits >= (lo << 5), xv, jnp.float32(0.0))


def _sc_part(x, k, d_off, d_span):
    n, hw1, d = x.shape
    nworkers = 32
    groups_per_worker = (n * d_span) // _NCH // nworkers
    mesh = plsc.VectorSubcoreMesh(core_axis_name="c", subcore_axis_name="s")

    def body(x_hbm, o_hbm, b0, b1, b2, b3, hist1, hist2,
             i0, i1, i2, i3, o0, o1, o2, o3):
        _sc_body(x_hbm, o_hbm, (b0, b1, b2, b3), hist1, hist2,
                 (i0, i1, i2, i3), (o0, o1, o2, o3),
                 k=k, d_off=d_off, d_span=d_span,
                 groups_per_worker=groups_per_worker)

    f = pl.kernel(
        body,
        mesh=mesh,
        out_type=jax.ShapeDtypeStruct((n, hw1, d_span), x.dtype),
        scratch_types=(
            [pltpu.VMEM((_CHUNK, _NCH), jnp.float32) for _ in range(_NBUF)]
            + [pltpu.VMEM((_NB1 * 16,), jnp.int32),
               pltpu.VMEM((_NB2 * 16,), jnp.int32)]
            + [pltpu.SemaphoreType.DMA for _ in range(2 * _NBUF)]
        ),
        compiler_params=pltpu.CompilerParams(
            use_tc_tiling_on_sc=False, needs_layout_passes=False),
    )
    return f(x)


def _tc_part(x, k, d_span):
    n, hw1, d = x.shape
    cblk = min(d_span, 256)
    return pl.pallas_call(
        functools.partial(_tc_select_body, k=k),
        grid=(n, d_span // cblk),
        in_specs=[pl.BlockSpec((1, hw1, cblk), lambda i, j: (i, 0, j))],
        out_specs=pl.BlockSpec((1, hw1, cblk), lambda i, j: (i, 0, j)),
        out_shape=jax.ShapeDtypeStruct((n, hw1, d_span), x.dtype),
        compiler_params=pltpu.CompilerParams(
            dimension_semantics=("parallel", "parallel"),
        ),
    )(x)


def _concat_body(a_ref, b_ref, o_ref, *, ja):
    j = pl.program_id(1)

    @pl.when(j < ja)
    def _():
        o_ref[...] = a_ref[...]

    @pl.when(j >= ja)
    def _():
        o_ref[...] = b_ref[...]


def _tc_concat(a, b):
    n, hw1, da = a.shape
    db = b.shape[2]
    cblk = 256
    ja, jb = da // cblk, db // cblk
    return pl.pallas_call(
        functools.partial(_concat_body, ja=ja),
        grid=(n, ja + jb),
        in_specs=[
            pl.BlockSpec((1, hw1, cblk),
                         lambda i, j: (i, 0, jnp.minimum(j, ja - 1))),
            pl.BlockSpec((1, hw1, cblk),
                         lambda i, j: (i, 0, jnp.maximum(j - ja, 0))),
        ],
        out_specs=pl.BlockSpec((1, hw1, cblk), lambda i, j: (i, 0, j)),
        out_shape=jax.ShapeDtypeStruct((n, hw1, da + db), a.dtype),
        compiler_params=pltpu.CompilerParams(
            dimension_semantics=("parallel", "arbitrary"),
        ),
    )(a, b)


_TC_SPAN = 512


def kernel(x):
    n, hw1, d = x.shape
    k = max(1, int(hw1 * _TOPK_FRAC))
    tc_span = _TC_SPAN if 0 < _TC_SPAN < d else 0
    sc_span = d - tc_span
    if tc_span == 0:
        return _sc_part(x, k, 0, sc_span)
    tc_out = _tc_part(x, k, tc_span)
    sc_out = _sc_part(
        jax.lax.slice_in_dim(x, tc_span, d, axis=2), k, 0, sc_span)
    return _tc_concat(tc_out, sc_out)
